# speedup vs baseline: 4.3508x; 4.3508x over previous
import functools

import jax
import jax.numpy as jnp
from jax import lax
from jax.experimental import pallas as pl
from jax.experimental.pallas import tpu as pltpu
from jax.experimental.pallas import tpu_sc as plsc

N_NODES = 10000
DIM = 128

NC = 2
NS = 16
NW = NC * NS

CHUNK = 128
ZC = 64
N_PAD = 10240
ROW_BLK = 400


def _matmul_y2(x, w2):
    def body(x_ref, w_ref, o_ref):
        o_ref[...] = jnp.dot(x_ref[...], w_ref[...],
                             preferred_element_type=jnp.float32)

    grid = N_NODES // ROW_BLK
    return pl.pallas_call(
        body,
        grid=(grid,),
        in_specs=[
            pl.BlockSpec((ROW_BLK, DIM), lambda i: (i, 0)),
            pl.BlockSpec((DIM, DIM), lambda i: (0, 0)),
        ],
        out_specs=pl.BlockSpec((ROW_BLK, DIM), lambda i: (i, 0)),
        out_shape=jax.ShapeDtypeStruct((N_NODES, DIM), jnp.float32),
    )(x, w2)


def _make_sc_scatter(n_chunks):
    edges_per_tile = n_chunks * CHUNK
    rows_per_tile = N_PAD // NS
    mesh = plsc.VectorSubcoreMesh(core_axis_name="c", subcore_axis_name="s")

    @functools.partial(
        pl.kernel,
        mesh=mesh,
        out_type=jax.ShapeDtypeStruct((NC, N_PAD, DIM), jnp.float32),
        scratch_types=[
            pltpu.VMEM((ZC, DIM), jnp.float32),
            pltpu.VMEM((CHUNK,), jnp.int32),
            pltpu.VMEM((CHUNK,), jnp.int32),
            pltpu.VMEM((CHUNK, DIM), jnp.float32),
            pltpu.VMEM_SHARED((N_PAD, DIM), jnp.float32),
            pltpu.SemaphoreType.DMA,
        ],
    )
    def sc_kernel(y2_hbm, src_hbm, dst_hbm, out_hbm,
                  zbuf, sidx, didx, rows, acc, sem):
        cid = lax.axis_index("c")
        sid = lax.axis_index("s")
        wid = sid * NC + cid

        def zrow(i, carry):
            for j in range(DIM // 16):
                zbuf[i, pl.ds(j * 16, 16)] = jnp.zeros((16,), jnp.float32)
            return carry
        lax.fori_loop(0, ZC, zrow, 0)

        def zcopy(i, carry):
            pltpu.sync_copy(
                zbuf, acc.at[pl.ds(sid * rows_per_tile + i * ZC, ZC)])
            return carry
        lax.fori_loop(0, rows_per_tile // ZC, zcopy, 0)
        plsc.subcore_barrier()

        base0 = wid * edges_per_tile

        def chunk_body(g, carry):
            base = base0 + g * CHUNK
            pltpu.sync_copy(src_hbm.at[pl.ds(base, CHUNK)], sidx)
            pltpu.sync_copy(dst_hbm.at[pl.ds(base, CHUNK)], didx)
            pltpu.async_copy(y2_hbm.at[sidx], rows, sem).wait()
            pltpu.sync_copy(rows, acc.at[didx], add=True)
            return carry
        lax.fori_loop(0, n_chunks, chunk_body, 0)
        plsc.subcore_barrier()

        lo = sid * rows_per_tile
        pltpu.sync_copy(acc.at[pl.ds(lo, rows_per_tile)],
                        out_hbm.at[cid, pl.ds(lo, rows_per_tile)])

    return sc_kernel


def _final(x, w1, partials):
    def body(x_ref, w_ref, p_ref, o_ref):
        acc = jnp.dot(x_ref[...], w_ref[...],
                      preferred_element_type=jnp.float32)
        acc = acc + p_ref[0] + p_ref[1]
        o_ref[...] = jnp.maximum(acc, 0.0)

    grid = N_NODES // ROW_BLK
    return pl.pallas_call(
        body,
        grid=(grid,),
        in_specs=[
            pl.BlockSpec((ROW_BLK, DIM), lambda i: (i, 0)),
            pl.BlockSpec((DIM, DIM), lambda i: (0, 0)),
            pl.BlockSpec((NC, ROW_BLK, DIM), lambda i: (0, i, 0)),
        ],
        out_specs=pl.BlockSpec((ROW_BLK, DIM), lambda i: (i, 0)),
        out_shape=jax.ShapeDtypeStruct((N_NODES, DIM), jnp.float32),
    )(x, w1, partials)


def kernel(x, edge_index, W1, W2):
    src = edge_index[0].astype(jnp.int32)
    dst = edge_index[1].astype(jnp.int32)
    n_edges = src.shape[0]
    per = NW * CHUNK
    n_chunks = -(-n_edges // per)
    e_pad = n_chunks * per
    pad = e_pad - n_edges
    if pad:
        src = jnp.concatenate([src, jnp.zeros((pad,), jnp.int32)])
        dst = jnp.concatenate([dst, jnp.full((pad,), N_NODES, jnp.int32)])

    y2 = _matmul_y2(x, W2)
    partials = _make_sc_scatter(n_chunks)(y2, src, dst)
    return _final(x, W1, partials)
